# baseline (speedup 1.0000x reference)
import functools

import jax
import jax.numpy as jnp
from jax import lax
from jax.experimental import pallas as pl
from jax.experimental.pallas import tpu as pltpu
from jax.experimental.pallas import tpu_sc as plsc

NG = 64
D = 128
N = 10000
NP = 10240
NCH = NP // 128
E = 320000
NC, NS = 2, 16
NW = NC * NS
NPT = NP // NW
CHK = 64
EPF = E + 512
EPAD = EPF + 8 * CHK
ZROW = NP - 8

BOUNDS = (10080, 20160, 30240, 40320, 50400, 60480, 70560, 80640, 90720,
          100800, 110880, 120720, 130560, 140400, 150240, 160000, 170080,
          180160, 190240, 200320, 210400, 220480, 230560, 240640, 250720,
          260800, 270880, 280720, 290560, 300400, 310240)


def _sc_agg_body(x_hbm, srcf_hbm, enc_hbm, zeros_hbm, tb_hbm, out_hbm,
                 i0, e0, r0, i1, e1, r1, outbuf, tbv, se0, se1, sem0, sem1):
  cid = lax.axis_index("c")
  sid = lax.axis_index("s")
  w = cid * NS + sid
  pltpu.sync_copy(tb_hbm, tbv)
  pltpu.sync_copy(zeros_hbm, outbuf)
  trow = tbv[w, :]
  e_lo = trow[0]
  e_hi = trow[1]
  e_al = trow[2]
  nch2 = trow[3]
  node0 = w * NPT

  bufs = ((i0, e0, r0, se0, sem0), (i1, e1, r1, se1, sem1))

  def start_chunk(b, j):
    ib, eb, rb, seb, smb = bufs[b]
    base = pl.multiple_of(e_al + j * CHK, 8)
    pltpu.sync_copy(srcf_hbm.at[pl.ds(base, CHK)], ib)
    pltpu.sync_copy(enc_hbm.at[pl.ds(base, CHK)], eb)
    pltpu.async_copy(x_hbm.at[ib], rb, smb)
    for gi in range(CHK // 16):
      vec = eb[pl.ds(gi * 16, 16)]
      for l in range(16):
        seb[gi * 16 + l] = vec[l]

  for b in (0, 1):
    start_chunk(b, b)

  def outer(g, carry):
    for b in (0, 1):
      j = 2 * g + b
      ib, eb, rb, seb, smb = bufs[b]
      base = e_al + j * CHK
      pltpu.make_async_copy(x_hbm.at[ib], rb, smb).wait()
      s = jnp.maximum(e_lo, base)
      t = jnp.minimum(e_hi, base + CHK)

      def inner(ei, c):
        a0, a1_, a2, a3, a4, a5, a6, a7, prev = c
        i = ei - base
        enc_e = seb[i]
        node = enc_e & 0xFFFF
        brk = enc_e >> 16
        rel = node - node0
        fl = (rel != prev) | (brk != 0)

        @pl.when(fl & (prev >= 0))
        def _():
          for k, ak in enumerate((a0, a1_, a2, a3, a4, a5, a6, a7)):
            outbuf[prev, pl.ds(k * 16, 16)] = (
                outbuf[prev, pl.ds(k * 16, 16)] + ak)

        rows = [rb[i, pl.ds(k * 16, 16)] for k in range(8)]
        acc = [jnp.where(fl, rows[k], ak + rows[k])
               for k, ak in enumerate((a0, a1_, a2, a3, a4, a5, a6, a7))]
        return (*acc, rel)

      carry = lax.fori_loop(s, t, inner, carry)
      start_chunk(b, j + 2)
    return carry

  zero = jnp.zeros((16,), jnp.float32)
  init = (zero,) * 8 + (jnp.int32(-1),)
  fin = lax.fori_loop(0, nch2, outer, init)
  for b in (0, 1):
    ib, eb, rb, seb, smb = bufs[b]
    pltpu.make_async_copy(x_hbm.at[ib], rb, smb).wait()
  prev = fin[8]

  @pl.when(prev >= 0)
  def _():
    for k in range(8):
      outbuf[prev, pl.ds(k * 16, 16)] = (
          outbuf[prev, pl.ds(k * 16, 16)] + fin[k])

  pltpu.sync_copy(outbuf, out_hbm.at[pl.ds(node0, NPT)])


@functools.cache
def _make_sc_agg():
  return pl.kernel(
      _sc_agg_body,
      out_type=jax.ShapeDtypeStruct((NP, D), jnp.float32),
      mesh=plsc.VectorSubcoreMesh(core_axis_name="c", subcore_axis_name="s",
                                  num_cores=NC, num_subcores=NS),
      scratch_types=[
          pltpu.VMEM((CHK,), jnp.int32),
          pltpu.VMEM((CHK,), jnp.int32),
          pltpu.VMEM((CHK, D), jnp.float32),
          pltpu.VMEM((CHK,), jnp.int32),
          pltpu.VMEM((CHK,), jnp.int32),
          pltpu.VMEM((CHK, D), jnp.float32),
          pltpu.VMEM((NPT, D), jnp.float32),
          pltpu.VMEM((NW, 16), jnp.int32),
          pltpu.SMEM((CHK,), jnp.int32),
          pltpu.SMEM((CHK,), jnp.int32),
          pltpu.SemaphoreType.DMA,
          pltpu.SemaphoreType.DMA,
      ],
  )


def _dense_body(ag, xr, wrel, wroot, brel, gam, bet, zo):
  acc = jnp.dot(ag[...], wrel[...], preferred_element_type=jnp.float32)
  acc = acc + brel[...]
  acc = acc + jnp.dot(xr[...], wroot[...], preferred_element_type=jnp.float32)
  z = jnp.maximum(acc, 0.0)
  zo[...] = z / jnp.sqrt(jnp.float32(1.0 + 1e-5)) * gam[...] + bet[...]


_dense = pl.pallas_call(
    _dense_body,
    grid=(NCH,),
    in_specs=[
        pl.BlockSpec((128, D), lambda i: (i, 0)),
        pl.BlockSpec((128, D), lambda i: (i, 0)),
        pl.BlockSpec((D, D), lambda i: (0, 0)),
        pl.BlockSpec((D, D), lambda i: (0, 0)),
        pl.BlockSpec((1, D), lambda i: (0, 0)),
        pl.BlockSpec((1, D), lambda i: (0, 0)),
        pl.BlockSpec((1, D), lambda i: (0, 0)),
    ],
    out_specs=pl.BlockSpec((128, D), lambda i: (i, 0)),
    out_shape=jax.ShapeDtypeStruct((NP, D), jnp.float32),
)


def _nt(a, b):
  return lax.dot_general(a, b, (((1,), (1,)), ((), ())),
                         preferred_element_type=jnp.float32)


def _pool_body(z, valid, posv, batchv, onehot, onehotT, pvec, racc,
               glo, ghi, culo, cuhi,
               xo, valo, poso, racco, score_s):
  pv = pvec[...]
  nrm = jnp.sqrt(jnp.sum(pv * pv, axis=1, keepdims=True))

  def ph_a(c, carry):
    zc = z[pl.ds(c * 128, 128), :]
    s = _nt(pv, zc)
    score_s[pl.ds(c, 1), :] = jnp.tanh(s / nrm)
    return carry

  lax.fori_loop(0, NCH, ph_a, 0)

  def ph_b(c, acc):
    vrow = valid[pl.ds(c, 1), :]
    oc = onehot[pl.ds(c * 128, 128), :]
    return acc + jnp.dot(vrow, oc, preferred_element_type=jnp.float32)

  c64 = lax.fori_loop(0, NCH, ph_b, jnp.zeros((1, NG), jnp.float32))
  k64 = jnp.ceil(0.5 * c64)
  ii = lax.broadcasted_iota(jnp.int32, (NG, NG), 0)
  jj = lax.broadcasted_iota(jnp.int32, (NG, NG), 1)
  mlt = jnp.where(ii < jj, 1.0, 0.0).astype(jnp.float32)
  start64 = jnp.dot(c64, mlt, preferred_element_type=jnp.float32)

  i0 = lax.broadcasted_iota(jnp.int32, (128, 128), 0)
  i1 = lax.broadcasted_iota(jnp.int32, (128, 128), 1)
  ident = jnp.where(i0 == i1, 1.0, 0.0).astype(jnp.float32)

  def ph_c(c, carry):
    gapacc, cntacc = carry
    srow = score_s[pl.ds(c, 1), :]
    brow = batchv[pl.ds(c, 1), :]
    prow = posv[pl.ds(c, 1), :]
    vrow = valid[pl.ds(c, 1), :]

    def ph_u(u, acc):
      su = score_s[pl.ds(u, 1), :]
      bu = batchv[pl.ds(u, 1), :]
      pu = posv[pl.ds(u, 1), :]
      vu = valid[pl.ds(u, 1), :]
      su_c = _nt(ident, su)
      bu_c = _nt(ident, bu)
      pu_c = _nt(ident, pu)
      vu_c = _nt(ident, vu)
      beq = bu_c == brow
      sgt = su_c > srow
      seq = su_c == srow
      plt = pu_c < prow
      cond = beq & (vu_c > 0.5) & (sgt | (seq & plt))
      return acc + jnp.where(cond, 1.0, 0.0)

    acc_t = lax.fori_loop(culo[c], cuhi[c], ph_u,
                          jnp.zeros((128, 128), jnp.float32))
    rrow = jnp.sum(acc_t, axis=0, keepdims=True)
    oc = onehot[pl.ds(c * 128, 128), :]
    krow = _nt(k64, oc)
    strow = _nt(start64, oc)
    sel = jnp.where((rrow < krow) & (vrow > 0.5), 1.0, 0.0)
    valo[pl.ds(c, 1), :] = sel
    poso[pl.ds(c, 1), :] = strow + rrow
    sel_c = _nt(ident, sel)
    s_c = _nt(ident, srow)
    zc = z[pl.ds(c * 128, 128), :]
    xn = jnp.where(sel_c > 0.5, zc * s_c, 0.0)
    xo[pl.ds(c * 128, 128), :] = xn
    otc = onehotT[:, pl.ds(c * 128, 128)]
    gapacc = gapacc + jnp.dot(otc, xn, preferred_element_type=jnp.float32)
    cntacc = cntacc + jnp.dot(sel, oc, preferred_element_type=jnp.float32)
    return (gapacc, cntacc)

  gap_sum, cnt = lax.fori_loop(
      0, NCH, ph_c,
      (jnp.zeros((NG, D), jnp.float32), jnp.zeros((1, NG), jnp.float32)))

  gi = lax.broadcasted_iota(jnp.int32, (NG, NG), 0)
  gj = lax.broadcasted_iota(jnp.int32, (NG, NG), 1)
  ident_g = jnp.where(gi == gj, 1.0, 0.0).astype(jnp.float32)
  cnt_col = _nt(ident_g, cnt)
  gap = gap_sum / jnp.maximum(cnt_col, 1.0)

  neg_inf = jnp.float32(-jnp.inf)
  for g in range(NG):
    def ph_g(cc, m):
      xc = xo[pl.ds(cc * 128, 128), :]
      selrow = valo[pl.ds(cc, 1), :]
      brow = batchv[pl.ds(cc, 1), :]
      mrow = jnp.where((brow == jnp.float32(g)) & (selrow > 0.5), 1.0, 0.0)
      mcol = _nt(ident, mrow)
      xm = jnp.where(mcol > 0.5, xc, neg_inf)
      return jnp.maximum(m, jnp.max(xm, axis=0, keepdims=True))

    gmax = lax.fori_loop(glo[g], ghi[g], ph_g,
                         jnp.full((1, D), neg_inf, jnp.float32))
    racco[g:g + 1, 0:D] = racc[g:g + 1, 0:D] + gmax
    racco[g:g + 1, D:2 * D] = racc[g:g + 1, D:2 * D] + gap[g:g + 1, :]


_pool = pl.pallas_call(
    _pool_body,
    in_specs=[
        pl.BlockSpec(memory_space=pltpu.VMEM),
        pl.BlockSpec(memory_space=pltpu.VMEM),
        pl.BlockSpec(memory_space=pltpu.VMEM),
        pl.BlockSpec(memory_space=pltpu.VMEM),
        pl.BlockSpec(memory_space=pltpu.VMEM),
        pl.BlockSpec(memory_space=pltpu.VMEM),
        pl.BlockSpec(memory_space=pltpu.VMEM),
        pl.BlockSpec(memory_space=pltpu.VMEM),
        pl.BlockSpec(memory_space=pltpu.SMEM),
        pl.BlockSpec(memory_space=pltpu.SMEM),
        pl.BlockSpec(memory_space=pltpu.SMEM),
        pl.BlockSpec(memory_space=pltpu.SMEM),
    ],
    out_specs=[
        pl.BlockSpec(memory_space=pltpu.VMEM),
        pl.BlockSpec(memory_space=pltpu.VMEM),
        pl.BlockSpec(memory_space=pltpu.VMEM),
        pl.BlockSpec(memory_space=pltpu.VMEM),
    ],
    out_shape=[
        jax.ShapeDtypeStruct((NP, D), jnp.float32),
        jax.ShapeDtypeStruct((NCH, 128), jnp.float32),
        jax.ShapeDtypeStruct((NCH, 128), jnp.float32),
        jax.ShapeDtypeStruct((NG, 2 * D), jnp.float32),
    ],
    scratch_shapes=[pltpu.VMEM((NCH, 128), jnp.float32)],
)


def _mlp_body(r, w1, b1, w2, b2, w3, b3, out):
  h = jnp.dot(r[...], w1[...], preferred_element_type=jnp.float32) + b1[...]
  h = jnp.maximum(h, 0.0)
  h = jnp.dot(h, w2[...], preferred_element_type=jnp.float32) + b2[...]
  h = jnp.maximum(h, 0.0)
  lg = jnp.dot(h, w3[...], preferred_element_type=jnp.float32) + b3[...]
  m = jnp.max(lg, axis=1, keepdims=True)
  e = jnp.exp(lg - m)
  s = jnp.sum(e, axis=1, keepdims=True)
  out[...] = lg - m - jnp.log(s)


def _mlp(racc, p):
  return pl.pallas_call(
      _mlp_body,
      out_shape=jax.ShapeDtypeStruct((NG, 10), jnp.float32),
  )(racc, p["lin1_W"], p["lin1_b"].reshape(1, -1),
    p["lin2_W"], p["lin2_b"].reshape(1, -1),
    p["lin3_W"], p["lin3_b"].reshape(1, -1))


def _aggregate(xp, srcf, enc, zeros_npt, tb):
  return _make_sc_agg()(xp, srcf, enc, zeros_npt, tb)


@jax.jit
def _forward(x, edge_index, batch, params):
  f32 = jnp.float32
  i32 = jnp.int32
  xp = jnp.zeros((NP, D), f32).at[:N].set(x)
  batch_p = jnp.concatenate(
      [batch, jnp.full((NP - N,), NG - 1, jnp.int32)])
  bvf = batch_p.astype(f32).reshape(NCH, 128)
  valid = (jnp.arange(NP) < N).astype(f32).reshape(NCH, 128)
  pos = jnp.arange(NP, dtype=f32).reshape(NCH, 128)
  onehot = (batch_p[:, None] == jnp.arange(NG)[None, :]).astype(f32)
  onehot_t = onehot.T

  idxg = jnp.arange(NG, dtype=i32)
  lo = jnp.searchsorted(batch, idxg, side="left").astype(i32)
  hi = jnp.searchsorted(batch, idxg, side="right").astype(i32)
  glo = lo // 128
  ghi = jnp.where(hi > lo, (hi + 127) // 128, glo)
  cidx = jnp.arange(NCH)
  first_b = batch_p[cidx * 128]
  last_b = batch_p[cidx * 128 + 127]
  culo = (lo[first_b] // 128).astype(i32)
  cuhi = jnp.maximum((hi[last_b] + 127) // 128, culo).astype(i32)

  src0 = jnp.concatenate([edge_index[0], jnp.full((EPF - E,), ZROW, i32)])
  dst0 = jnp.concatenate([edge_index[1], jnp.full((EPF - E,), ZROW, i32)])
  order0 = jnp.argsort(dst0, stable=True)
  src_s = src0[order0]
  dst_s = dst0[order0]
  real_s = order0 < E
  estart = jnp.searchsorted(dst_s, jnp.arange(NP + 1, dtype=i32)).astype(i32)
  wgrid = jnp.searchsorted(
      dst_s, (jnp.arange(NW + 1, dtype=i32) * NPT)).astype(i32)
  w_lo = wgrid[:NW]
  w_hi = wgrid[1:]
  w_al = w_lo & ~jnp.int32(7)
  nch2 = (((w_hi - w_al + CHK - 1) // CHK) + 1) // 2
  tb = jnp.zeros((NW, 16), i32)
  tb = tb.at[:, 0].set(w_lo).at[:, 1].set(w_hi).at[:, 2].set(w_al)
  tb = tb.at[:, 3].set(nch2)

  barr = jnp.asarray(BOUNDS, dtype=i32)
  zeros_npt = jnp.zeros((NPT, D), f32)
  racc = jnp.zeros((NG, 2 * D), f32)
  eidx = jnp.arange(EPF)
  segstart_of_edge = estart[dst_s]

  for i in range(1, 6):
    vb = (valid.reshape(-1) > 0.5)
    posi = pos.reshape(-1).astype(i32)
    ev_u = vb[src0] & vb[dst0] & (eidx < E)
    ev_s = ev_u[order0]
    vcum0 = jnp.concatenate([jnp.zeros((1,), i32),
                             jnp.cumsum(ev_s.astype(i32))])
    vrank = vcum0[eidx] - vcum0[segstart_of_edge]
    vdeg = vcum0[estart[1:]] - vcum0[estart[:-1]]
    cnt_q = jnp.zeros((NP,), i32).at[posi].add(
        jnp.where(vb, vdeg, 0), mode="drop")
    n_inv = jnp.int32(E) - jnp.sum(ev_u.astype(i32))
    ce = jnp.concatenate([jnp.zeros((1,), i32),
                          jnp.cumsum(cnt_q)])
    invc0 = jnp.concatenate(
        [jnp.zeros((1,), i32),
         jnp.cumsum(((~ev_u) & (eidx < E)).astype(i32))])
    inv_s = invc0[order0]
    q_s = jnp.where(ev_s, posi[dst_s], 0)
    refpos = jnp.where(q_s > 0,
                       n_inv + ce[q_s] + vrank,
                       inv_s + vrank)
    sb = jnp.searchsorted(barr, refpos)
    brk = ev_s & (sb < len(BOUNDS)) & (barr[jnp.minimum(sb, len(BOUNDS) - 1)]
                                       == refpos)
    enc = (dst_s + (brk.astype(i32) << 16)).astype(i32)
    srcf = jnp.where(ev_s, src_s, ZROW).astype(i32)
    enc = jnp.concatenate([enc, jnp.full((EPAD - EPF,), ZROW, i32)])
    srcf = jnp.concatenate([srcf, jnp.full((EPAD - EPF,), ZROW, i32)])

    aggr = _aggregate(xp, srcf, enc, zeros_npt, tb)
    z = _dense(aggr, xp,
               params[f"conv{i}_Wrel"], params[f"conv{i}_Wroot"],
               params[f"conv{i}_brel"].reshape(1, D),
               params[f"bn{i}_gamma"].reshape(1, D),
               params[f"bn{i}_beta"].reshape(1, D))
    xp, valid, pos, racc = _pool(
        z, valid, pos, bvf, onehot, onehot_t,
        params[f"pool{i}_p"].reshape(1, D), racc,
        glo, ghi, culo, cuhi)

  return _mlp(racc, params)


def kernel(x, edge_index, batch, params):
  return _forward(x, edge_index, batch, params)
